# Initial kernel scaffold; baseline (speedup 1.0000x reference)
import jax
import jax.numpy as jnp
from jax.experimental import pallas as pl


def kernel(inputs, table):
    raise NotImplementedError("write your pallas kernel here")


# speedup vs baseline: 2.7501x; 2.7501x over previous
import jax
import jax.numpy as jnp
from jax.experimental import pallas as pl

MAXLEN = 8192
OUTPUT_DIM = 1024
BLOCK_ROWS = 512


def _copy_body(tab_ref, out_ref):
    out_ref[0] = tab_ref[...]


def kernel(inputs, table):
    del inputs
    grid = (MAXLEN // BLOCK_ROWS,)
    out = pl.pallas_call(
        _copy_body,
        grid=grid,
        in_specs=[pl.BlockSpec((BLOCK_ROWS, OUTPUT_DIM), lambda i: (i, 0))],
        out_specs=pl.BlockSpec((1, BLOCK_ROWS, OUTPUT_DIM), lambda i: (0, i, 0)),
        out_shape=jax.ShapeDtypeStruct((1, MAXLEN, OUTPUT_DIM), table.dtype),
    )(table)
    return out
